# baseline (speedup 1.0000x reference)
import functools

import jax
import jax.numpy as jnp
from jax import lax
from jax.experimental import pallas as pl
from jax.experimental.pallas import tpu as pltpu
from jax.experimental.pallas import tpu_sc as plsc

VOCAB = 1000000
D = 16
B = 16384
NC = 2
NS = 16
NW = NC * NS
PW = B // NW
CH = 128
NCH = PW // CH
GV = VOCAB // 8
LN2 = 0.6931471805599453


def _softplus(x):
    y = 1.0 + jnp.exp(x)
    bits = lax.bitcast_convert_type(y, jnp.int32)
    e = ((bits >> 23) - 127).astype(jnp.float32)
    m = lax.bitcast_convert_type((bits & 0x007FFFFF) | 0x3F800000, jnp.float32)
    s = (m - 1.0) / (m + 1.0)
    s2 = s * s
    logm = 2.0 * s * (1.0 + s2 * (jnp.float32(1.0 / 3.0) + s2 * 0.2))
    return e * jnp.float32(LN2) + logm


_MESH = plsc.VectorSubcoreMesh(
    core_axis_name="c", subcore_axis_name="s", num_cores=NC, num_subcores=NS
)


@functools.partial(
    pl.kernel,
    out_type=jax.ShapeDtypeStruct((NW, PW), jnp.float32),
    mesh=_MESH,
    scratch_types=[
        pltpu.VMEM((4, NCH, CH), jnp.int32),
        pltpu.VMEM((NCH, CH), jnp.int32),
        pltpu.VMEM((PW, CH), jnp.float32),
        pltpu.VMEM((D, PW), jnp.float32),
        pltpu.VMEM((D, PW), jnp.float32),
        pltpu.VMEM((D, PW), jnp.float32),
        pltpu.VMEM((D, PW), jnp.float32),
        pltpu.VMEM((PW,), jnp.float32),
        pltpu.SemaphoreType.DMA,
    ],
    compiler_params=pltpu.CompilerParams(needs_layout_passes=False),
)
def _sc_kernel(idx_hbm, ce_hbm, xe_hbm, out_hbm,
               idx_v, gidx, slab, ra, rb, rc, rd, ov, sem):
    wid = lax.axis_index("s") * NC + lax.axis_index("c")
    pltpu.sync_copy(idx_hbm.at[wid], idx_v)

    for t, (tbl, rows) in enumerate(
        ((ce_hbm, ra), (xe_hbm, rb), (ce_hbm, rc), (xe_hbm, rd))
    ):
        for ch in range(NCH):
            for g in range(CH // 16):
                v = idx_v[t, ch, pl.ds(g * 16, 16)]
                gidx[ch, pl.ds(g * 16, 16)] = v >> 3

        copies = [
            pltpu.async_copy(
                tbl.at[gidx.at[ch]], slab.at[pl.ds(ch * CH, CH)], sem
            )
            for ch in range(NCH)
        ]
        for cp in copies:
            cp.wait()

        def ext_body(blk, carry, t=t, rows=rows):
            off = blk * 16
            jrows = off + lax.iota(jnp.int32, 16)
            w = plsc.load_gather(
                idx_v, [jnp.full((16,), t, jnp.int32), jrows // CH, jrows % CH]
            )
            base = (w & 7) * D
            for d in range(D):
                rows[d, pl.ds(off, 16)] = plsc.load_gather(
                    slab, [jrows, base + d]
                )
            return carry

        lax.fori_loop(0, PW // 16, ext_body, 0)

    def blk_body(blk, carry):
        off = blk * 16
        accp = jnp.zeros((16,), jnp.float32)
        accn = jnp.zeros((16,), jnp.float32)
        for d in range(D):
            accp += ra[d, pl.ds(off, 16)] * rb[d, pl.ds(off, 16)]
            accn += rc[d, pl.ds(off, 16)] * rd[d, pl.ds(off, 16)]
        p = jnp.clip(accp, -10.0, 10.0)
        n = jnp.clip(accn, -10.0, 10.0)
        ov[pl.ds(off, 16)] = _softplus(-p) + _softplus(n)
        return carry

    lax.fori_loop(0, PW // 16, blk_body, 0)
    pltpu.sync_copy(ov, out_hbm.at[wid])


def kernel(central_word, context_word, neg_central_word, neg_context_word,
           central_embeddings, context_embeddings):
    idx = jnp.stack(
        [central_word, context_word, neg_central_word, neg_context_word]
    ).astype(jnp.int32)
    idx = idx.reshape(4, NW, NCH, CH).transpose(1, 0, 2, 3)
    ce8 = central_embeddings.reshape(GV, 8 * D)
    xe8 = context_embeddings.reshape(GV, 8 * D)
    out = _sc_kernel(idx, ce8, xe8)
    return out.reshape(B)


# speedup vs baseline: 1.2096x; 1.2096x over previous
import functools

import jax
import jax.numpy as jnp
from jax import lax
from jax.experimental import pallas as pl
from jax.experimental.pallas import tpu as pltpu
from jax.experimental.pallas import tpu_sc as plsc

VOCAB = 1000000
D = 16
B = 16384
NC = 2
NS = 16
NW = NC * NS
PW = B // NW
CH = 128
NCH = PW // CH
LN2 = 0.6931471805599453


def _softplus(x):
    y = 1.0 + jnp.exp(x)
    bits = lax.bitcast_convert_type(y, jnp.int32)
    e = ((bits >> 23) - 127).astype(jnp.float32)
    m = lax.bitcast_convert_type((bits & 0x007FFFFF) | 0x3F800000, jnp.float32)
    s = (m - 1.0) / (m + 1.0)
    s2 = s * s
    logm = 2.0 * s * (1.0 + s2 * (jnp.float32(1.0 / 3.0) + s2 * 0.2))
    return e * jnp.float32(LN2) + logm


_MESH = plsc.VectorSubcoreMesh(
    core_axis_name="c", subcore_axis_name="s", num_cores=NC, num_subcores=NS
)


@functools.partial(
    pl.kernel,
    out_type=jax.ShapeDtypeStruct((NW, PW), jnp.float32),
    mesh=_MESH,
    scratch_types=[
        pltpu.VMEM((4, NCH, CH), jnp.int32),
        pltpu.VMEM((D, PW), jnp.float32),
        pltpu.VMEM((D, PW), jnp.float32),
        pltpu.VMEM((D, PW), jnp.float32),
        pltpu.VMEM((D, PW), jnp.float32),
        pltpu.VMEM((PW,), jnp.float32),
        pltpu.SemaphoreType.DMA,
    ],
    compiler_params=pltpu.CompilerParams(
        needs_layout_passes=False, use_tc_tiling_on_sc=False
    ),
)
def _sc_kernel(idx_hbm, *refs):
    ce_planes = refs[:D]
    xe_planes = refs[D:2 * D]
    out_hbm = refs[2 * D]
    idx_v, ra, rb, rc, rd, ov, sem = refs[2 * D + 1:]

    wid = lax.axis_index("s") * NC + lax.axis_index("c")
    pltpu.sync_copy(idx_hbm.at[wid], idx_v)

    copies = []
    for t, (planes, dst) in enumerate(
        ((ce_planes, ra), (xe_planes, rb), (ce_planes, rc), (xe_planes, rd))
    ):
        for ch in range(NCH):
            for d in range(D):
                copies.append(
                    pltpu.async_copy(
                        planes[d].at[idx_v.at[t, ch]],
                        dst.at[d, pl.ds(ch * CH, CH)],
                        sem,
                    )
                )
    for cp in copies:
        cp.wait()

    def blk_body(blk, carry):
        off = blk * 16
        accp = jnp.zeros((16,), jnp.float32)
        accn = jnp.zeros((16,), jnp.float32)
        for d in range(D):
            accp += ra[d, pl.ds(off, 16)] * rb[d, pl.ds(off, 16)]
            accn += rc[d, pl.ds(off, 16)] * rd[d, pl.ds(off, 16)]
        p = jnp.clip(accp, -10.0, 10.0)
        n = jnp.clip(accn, -10.0, 10.0)
        ov[pl.ds(off, 16)] = _softplus(-p) + _softplus(n)
        return carry

    lax.fori_loop(0, PW // 16, blk_body, 0)
    pltpu.sync_copy(ov, out_hbm.at[wid])


def kernel(central_word, context_word, neg_central_word, neg_context_word,
           central_embeddings, context_embeddings):
    idx = jnp.stack(
        [central_word, context_word, neg_central_word, neg_context_word]
    ).astype(jnp.int32)
    idx = idx.reshape(4, NW, NCH, CH).transpose(1, 0, 2, 3)
    def _planes(tbl):
        out = []
        for g in range(0, D, 4):
            out += list(lax.optimization_barrier(
                tuple(tbl[:, d] for d in range(g, g + 4))
            ))
        return out

    ce_planes = _planes(central_embeddings)
    xe_planes = _planes(context_embeddings)
    out = _sc_kernel(idx, *ce_planes, *xe_planes)
    return out.reshape(B)


# speedup vs baseline: 4.1899x; 3.4640x over previous
import functools

import jax
import jax.numpy as jnp
from jax import lax
from jax.experimental import pallas as pl
from jax.experimental.pallas import tpu as pltpu
from jax.experimental.pallas import tpu_sc as plsc

VOCAB = 1000000
D = 16
B = 16384
NC = 2
NS = 16
NW = NC * NS
PW = B // NW
CH = 128
NCH = PW // CH
LN2 = 0.6931471805599453


def _softplus(x):
    y = 1.0 + jnp.exp(x)
    bits = lax.bitcast_convert_type(y, jnp.int32)
    e = ((bits >> 23) - 127).astype(jnp.float32)
    m = lax.bitcast_convert_type((bits & 0x007FFFFF) | 0x3F800000, jnp.float32)
    s = (m - 1.0) / (m + 1.0)
    s2 = s * s
    logm = 2.0 * s * (1.0 + s2 * (jnp.float32(1.0 / 3.0) + s2 * 0.2))
    return e * jnp.float32(LN2) + logm


_MESH = plsc.VectorSubcoreMesh(
    core_axis_name="c", subcore_axis_name="s", num_cores=NC, num_subcores=NS
)


@functools.partial(
    pl.kernel,
    out_type=jax.ShapeDtypeStruct((NW, PW), jnp.float32),
    mesh=_MESH,
    scratch_types=[
        pltpu.VMEM((4, NCH, CH), jnp.int32),
        pltpu.VMEM((D, PW), jnp.float32),
        pltpu.VMEM((D, PW), jnp.float32),
        pltpu.VMEM((D, PW), jnp.float32),
        pltpu.VMEM((D, PW), jnp.float32),
        pltpu.VMEM((PW,), jnp.float32),
        pltpu.SemaphoreType.DMA,
    ],
    compiler_params=pltpu.CompilerParams(
        needs_layout_passes=False, use_tc_tiling_on_sc=False
    ),
)
def _sc_kernel(idx_hbm, *refs):
    ce_planes = refs[:D]
    xe_planes = refs[D:2 * D]
    out_hbm = refs[2 * D]
    idx_v, ra, rb, rc, rd, ov, sem = refs[2 * D + 1:]

    wid = lax.axis_index("s") * NC + lax.axis_index("c")
    pltpu.sync_copy(idx_hbm.at[wid], idx_v)

    copies = []
    for t, (planes, dst) in enumerate(
        ((ce_planes, ra), (xe_planes, rb), (ce_planes, rc), (xe_planes, rd))
    ):
        for ch in range(NCH):
            for d in range(D):
                copies.append(
                    pltpu.async_copy(
                        planes[d].at[idx_v.at[t, ch]],
                        dst.at[d, pl.ds(ch * CH, CH)],
                        sem,
                    )
                )
    for cp in copies:
        cp.wait()

    def blk_body(blk, carry):
        off = blk * 16
        accp = jnp.zeros((16,), jnp.float32)
        accn = jnp.zeros((16,), jnp.float32)
        for d in range(D):
            accp += ra[d, pl.ds(off, 16)] * rb[d, pl.ds(off, 16)]
            accn += rc[d, pl.ds(off, 16)] * rd[d, pl.ds(off, 16)]
        p = jnp.clip(accp, -10.0, 10.0)
        n = jnp.clip(accn, -10.0, 10.0)
        ov[pl.ds(off, 16)] = _softplus(-p) + _softplus(n)
        return carry

    lax.fori_loop(0, PW // 16, blk_body, 0)
    pltpu.sync_copy(ov, out_hbm.at[wid])


_VCH = 8192
_VGRID = -(-VOCAB // _VCH)


def _tc_planes_body(ce_ref, xe_ref, *out_refs):
    for d in range(D):
        out_refs[d][...] = ce_ref[d, :]
        out_refs[D + d][...] = xe_ref[d, :]


_tc_planes = pl.pallas_call(
    _tc_planes_body,
    grid=(_VGRID,),
    in_specs=[
        pl.BlockSpec((D, _VCH), lambda i: (0, i)),
        pl.BlockSpec((D, _VCH), lambda i: (0, i)),
    ],
    out_specs=[pl.BlockSpec((_VCH,), lambda i: (i,)) for _ in range(2 * D)],
    out_shape=[jax.ShapeDtypeStruct((VOCAB,), jnp.float32)] * (2 * D),
)


def kernel(central_word, context_word, neg_central_word, neg_context_word,
           central_embeddings, context_embeddings):
    idx = jnp.stack(
        [central_word, context_word, neg_central_word, neg_context_word]
    ).astype(jnp.int32)
    idx = idx.reshape(4, NW, NCH, CH).transpose(1, 0, 2, 3)
    planes = _tc_planes(central_embeddings.T, context_embeddings.T)
    out = _sc_kernel(idx, *planes)
    return out.reshape(B)


# speedup vs baseline: 5.8443x; 1.3949x over previous
import functools

import jax
import jax.numpy as jnp
from jax import lax
from jax.experimental import pallas as pl
from jax.experimental.pallas import tpu as pltpu
from jax.experimental.pallas import tpu_sc as plsc

VOCAB = 1000000
D = 16
B = 16384
NC = 2
NS = 16
NW = NC * NS
PW = B // NW
CH = 128
NCH = PW // CH
LN2 = 0.6931471805599453


def _softplus(x):
    y = 1.0 + jnp.exp(x)
    bits = lax.bitcast_convert_type(y, jnp.int32)
    e = ((bits >> 23) - 127).astype(jnp.float32)
    m = lax.bitcast_convert_type((bits & 0x007FFFFF) | 0x3F800000, jnp.float32)
    s = (m - 1.0) / (m + 1.0)
    s2 = s * s
    logm = 2.0 * s * (1.0 + s2 * (jnp.float32(1.0 / 3.0) + s2 * 0.2))
    return e * jnp.float32(LN2) + logm


_MESH = plsc.VectorSubcoreMesh(
    core_axis_name="c", subcore_axis_name="s", num_cores=NC, num_subcores=NS
)


@functools.partial(
    pl.kernel,
    out_type=jax.ShapeDtypeStruct((NW, PW), jnp.float32),
    mesh=_MESH,
    scratch_types=[
        pltpu.VMEM((4, NCH, CH), jnp.int32),
        pltpu.VMEM((D, PW), jnp.float32),
        pltpu.VMEM((D, PW), jnp.float32),
        pltpu.VMEM((D, PW), jnp.float32),
        pltpu.VMEM((D, PW), jnp.float32),
        pltpu.VMEM((PW,), jnp.float32),
        pltpu.SemaphoreType.DMA,
    ],
    compiler_params=pltpu.CompilerParams(
        needs_layout_passes=False, use_tc_tiling_on_sc=False
    ),
)
def _sc_kernel(idx_hbm, *refs):
    ce_planes = refs[:D]
    xe_planes = refs[D:2 * D]
    out_hbm = refs[2 * D]
    idx_v, ra, rb, rc, rd, ov, sem = refs[2 * D + 1:]

    wid = lax.axis_index("s") * NC + lax.axis_index("c")
    pltpu.sync_copy(idx_hbm.at[wid], idx_v)

    copies = []
    for t, (planes, dst) in enumerate(
        ((ce_planes, ra), (xe_planes, rb), (ce_planes, rc), (xe_planes, rd))
    ):
        for ch in range(NCH):
            for d in range(D):
                copies.append(
                    pltpu.async_copy(
                        planes[d].at[idx_v.at[t, ch]],
                        dst.at[d, pl.ds(ch * CH, CH)],
                        sem,
                    )
                )
    for cp in copies:
        cp.wait()

    def blk_body(blk, carry):
        off = blk * 16
        accp = jnp.zeros((16,), jnp.float32)
        accn = jnp.zeros((16,), jnp.float32)
        for d in range(D):
            accp += ra[d, pl.ds(off, 16)] * rb[d, pl.ds(off, 16)]
            accn += rc[d, pl.ds(off, 16)] * rd[d, pl.ds(off, 16)]
        p = jnp.clip(accp, -10.0, 10.0)
        n = jnp.clip(accn, -10.0, 10.0)
        ov[pl.ds(off, 16)] = _softplus(-p) + _softplus(n)
        return carry

    lax.fori_loop(0, PW // 16, blk_body, 0)
    pltpu.sync_copy(ov, out_hbm.at[wid])


_VCH = 32768
_VGRID = -(-VOCAB // _VCH)


def _tc_planes_body(ce_ref, xe_ref, *out_refs):
    for d in range(D):
        out_refs[d][...] = ce_ref[d, :]
        out_refs[D + d][...] = xe_ref[d, :]


_tc_planes = pl.pallas_call(
    _tc_planes_body,
    grid=(_VGRID,),
    in_specs=[
        pl.BlockSpec((D, _VCH), lambda i: (0, i)),
        pl.BlockSpec((D, _VCH), lambda i: (0, i)),
    ],
    out_specs=[pl.BlockSpec((_VCH,), lambda i: (i,)) for _ in range(2 * D)],
    out_shape=[jax.ShapeDtypeStruct((VOCAB,), jnp.float32)] * (2 * D),
)


def kernel(central_word, context_word, neg_central_word, neg_context_word,
           central_embeddings, context_embeddings):
    idx = jnp.stack(
        [central_word, context_word, neg_central_word, neg_context_word]
    ).astype(jnp.int32)
    idx = idx.reshape(4, NW, NCH, CH).transpose(1, 0, 2, 3)
    planes = _tc_planes(central_embeddings.T, context_embeddings.T)
    out = _sc_kernel(idx, *planes)
    return out.reshape(B)


# speedup vs baseline: 5.9316x; 1.0149x over previous
import functools

import jax
import jax.numpy as jnp
from jax import lax
from jax.experimental import pallas as pl
from jax.experimental.pallas import tpu as pltpu
from jax.experimental.pallas import tpu_sc as plsc

VOCAB = 1000000
D = 16
B = 16384
NC = 2
NS = 16
NW = NC * NS
PW = B // NW
CH = 128
NCH = PW // CH
LN2 = 0.6931471805599453


def _softplus(x):
    y = 1.0 + jnp.exp(x)
    bits = lax.bitcast_convert_type(y, jnp.int32)
    e = ((bits >> 23) - 127).astype(jnp.float32)
    m = lax.bitcast_convert_type((bits & 0x007FFFFF) | 0x3F800000, jnp.float32)
    s = (m - 1.0) / (m + 1.0)
    s2 = s * s
    logm = 2.0 * s * (1.0 + s2 * (jnp.float32(1.0 / 3.0) + s2 * 0.2))
    return e * jnp.float32(LN2) + logm


_MESH = plsc.VectorSubcoreMesh(
    core_axis_name="c", subcore_axis_name="s", num_cores=NC, num_subcores=NS
)


@functools.partial(
    pl.kernel,
    out_type=jax.ShapeDtypeStruct((NW, PW), jnp.float32),
    mesh=_MESH,
    scratch_types=[
        pltpu.VMEM((4, NCH, CH), jnp.int32),
        pltpu.VMEM((D, PW), jnp.float32),
        pltpu.VMEM((D, PW), jnp.float32),
        pltpu.VMEM((D, PW), jnp.float32),
        pltpu.VMEM((D, PW), jnp.float32),
        pltpu.VMEM((PW,), jnp.float32),
        pltpu.SemaphoreType.DMA,
    ],
    compiler_params=pltpu.CompilerParams(
        needs_layout_passes=False, use_tc_tiling_on_sc=False
    ),
)
def _sc_kernel(idx_hbm, *refs):
    ce_planes = refs[:D]
    xe_planes = refs[D:2 * D]
    out_hbm = refs[2 * D]
    idx_v, ra, rb, rc, rd, ov, sem = refs[2 * D + 1:]

    wid = lax.axis_index("s") * NC + lax.axis_index("c")
    pltpu.sync_copy(idx_hbm.at[wid], idx_v)

    copies = []
    for t, (planes, dst) in enumerate(
        ((ce_planes, ra), (xe_planes, rb), (ce_planes, rc), (xe_planes, rd))
    ):
        for ch in range(NCH):
            for d in range(D):
                copies.append(
                    pltpu.async_copy(
                        planes[d].at[idx_v.at[t, ch]],
                        dst.at[d, pl.ds(ch * CH, CH)],
                        sem,
                    )
                )
    for cp in copies:
        cp.wait()

    def blk_body(blk, carry):
        off = blk * 16
        accp = jnp.zeros((16,), jnp.float32)
        accn = jnp.zeros((16,), jnp.float32)
        for d in range(D):
            accp += ra[d, pl.ds(off, 16)] * rb[d, pl.ds(off, 16)]
            accn += rc[d, pl.ds(off, 16)] * rd[d, pl.ds(off, 16)]
        p = jnp.clip(accp, -10.0, 10.0)
        n = jnp.clip(accn, -10.0, 10.0)
        ov[pl.ds(off, 16)] = _softplus(-p) + _softplus(n)
        return carry

    lax.fori_loop(0, PW // 16, blk_body, 0)
    pltpu.sync_copy(ov, out_hbm.at[wid])


_VCH = 65536
_VGRID = -(-VOCAB // _VCH)


def _tc_planes_body(ce_ref, xe_ref, *out_refs):
    for d in range(D):
        out_refs[d][...] = ce_ref[d, :]
        out_refs[D + d][...] = xe_ref[d, :]


_tc_planes = pl.pallas_call(
    _tc_planes_body,
    grid=(_VGRID,),
    in_specs=[
        pl.BlockSpec((D, _VCH), lambda i: (0, i)),
        pl.BlockSpec((D, _VCH), lambda i: (0, i)),
    ],
    out_specs=[pl.BlockSpec((_VCH,), lambda i: (i,)) for _ in range(2 * D)],
    out_shape=[jax.ShapeDtypeStruct((VOCAB,), jnp.float32)] * (2 * D),
)


def kernel(central_word, context_word, neg_central_word, neg_context_word,
           central_embeddings, context_embeddings):
    idx = jnp.stack(
        [central_word, context_word, neg_central_word, neg_context_word]
    ).astype(jnp.int32)
    idx = idx.reshape(4, NW, NCH, CH).transpose(1, 0, 2, 3)
    planes = _tc_planes(central_embeddings.T, context_embeddings.T)
    out = _sc_kernel(idx, *planes)
    return out.reshape(B)
